# baseline (speedup 1.0000x reference)
import functools
import math

import jax
import jax.numpy as jnp
import numpy as np
from jax import lax
from jax.experimental import pallas as pl
from jax.experimental.pallas import tpu as pltpu
from jax.experimental.pallas import tpu_sc as plsc

BS = 2
NQ = 8192
EMBED = 256
HEADS = 8
LEVELS = 4
POINTS = 4
CH = EMBED // HEADS
SHAPES = [[128, 128], [64, 64], [32, 32], [16, 16]]
NV = sum(h * w for h, w in SHAPES)
NROWS_Q = NQ * BS
NOROWS = NQ * BS * HEADS
NSAMP = LEVELS * POINTS * 4
HLP = HEADS * LEVELS * POINTS

_lvl_of_col = np.array([(c // POINTS) % LEVELS for c in range(HLP)], np.int32)
_W_COL = np.array([SHAPES[l][1] for l in _lvl_of_col], np.float32)[None, :]
_H_COL = np.array([SHAPES[l][0] for l in _lvl_of_col], np.float32)[None, :]
_areas = [h * w for h, w in SHAPES]
_lstart = np.concatenate([[0], np.cumsum(_areas)[:-1]]).astype(np.int32)
_LS_COL = _lstart[_lvl_of_col][None, :]


def _mm_bias_body(x_ref, w_ref, b_ref, o_ref):
    o_ref[...] = (
        jnp.dot(x_ref[...], w_ref[...], preferred_element_type=jnp.float32)
        + b_ref[...]
    )


def _vproj_bh_body(x_ref, w_ref, b_ref, o_ref):
    x = x_ref[0]
    proj = jnp.dot(x, w_ref[...], preferred_element_type=jnp.float32) + b_ref[...]
    for h in range(HEADS):
        o_ref[0, h] = proj[:, h * CH:(h + 1) * CH].astype(jnp.bfloat16)


_A_BLOCK = 1280


def _tc_vproj_bh(value_t, w_val_t, b_val):
    grid = (BS, NV // _A_BLOCK)
    return pl.pallas_call(
        _vproj_bh_body,
        grid=grid,
        in_specs=[
            pl.BlockSpec((1, _A_BLOCK, EMBED), lambda b, i: (b, i, 0)),
            pl.BlockSpec((EMBED, EMBED), lambda b, i: (0, 0)),
            pl.BlockSpec((1, EMBED), lambda b, i: (0, 0)),
        ],
        out_specs=pl.BlockSpec((1, HEADS, _A_BLOCK, CH), lambda b, i: (b, 0, i, 0)),
        out_shape=jax.ShapeDtypeStruct((BS, HEADS, NV, CH), jnp.bfloat16),
    )(value_t, w_val_t, b_val.reshape(1, EMBED))


_P_BLOCK = 2048
_P_NB = NV * BS * HEADS // _P_BLOCK


def _pair_body(a_ref, b_ref, o_ref):
    a = a_ref[...]
    o_ref[:, 0:CH] = a
    o_ref[0:_P_BLOCK - 1, CH:2 * CH] = a[1:_P_BLOCK, :]
    o_ref[_P_BLOCK - 1:_P_BLOCK, CH:2 * CH] = b_ref[0:1, :]


def _tc_pair(table):
    return pl.pallas_call(
        _pair_body,
        grid=(_P_NB,),
        in_specs=[
            pl.BlockSpec((_P_BLOCK, CH), lambda i: (i, 0)),
            pl.BlockSpec((_P_BLOCK, CH),
                         lambda i: (jnp.minimum(i + 1, _P_NB - 1), 0)),
        ],
        out_specs=pl.BlockSpec((_P_BLOCK, 2 * CH), lambda i: (i, 0)),
        out_shape=jax.ShapeDtypeStruct((NV * BS * HEADS, 2 * CH), jnp.bfloat16),
    )(table, table)


def _mm_bias_res_body(x_ref, w_ref, b_ref, r_ref, o_ref):
    o_ref[...] = (
        jnp.dot(x_ref[...], w_ref[...], preferred_element_type=jnp.float32)
        + b_ref[...]
        + r_ref[...]
    )


def _sampling_body(q_ref, wox_ref, woy_ref, wa_ref, box_ref, boy_ref, ba_ref,
                   refx_ref, refy_ref, wl_ref, hl_ref, wli_ref, hli_ref,
                   ls_ref, oi_ref, ow_ref):
    q = q_ref[...]
    wl = wl_ref[...]
    hl = hl_ref[...]
    wli = wli_ref[...]
    hli = hli_ref[...]
    ls = ls_ref[...]

    offx = jnp.dot(q, wox_ref[...], preferred_element_type=jnp.float32) + box_ref[...]
    offy = jnp.dot(q, woy_ref[...], preferred_element_type=jnp.float32) + boy_ref[...]
    a = jnp.dot(q, wa_ref[...], preferred_element_type=jnp.float32) + ba_ref[...]

    x = refx_ref[...] * wl + offx - 0.5
    y = refy_ref[...] * hl + offy - 0.5
    x0 = jnp.floor(x)
    y0 = jnp.floor(y)
    fx = x - x0
    fy = y - y0
    x0i = x0.astype(jnp.int32)
    y0i = y0.astype(jnp.int32)

    vx0 = ((x0 >= 0.0) & (x0 <= wl - 1.0)).astype(jnp.float32)
    vx1 = ((x0 + 1.0 >= 0.0) & (x0 + 1.0 <= wl - 1.0)).astype(jnp.float32)
    vy0 = ((y0 >= 0.0) & (y0 <= hl - 1.0)).astype(jnp.float32)
    vy1 = ((y0 + 1.0 >= 0.0) & (y0 + 1.0 <= hl - 1.0)).astype(jnp.float32)

    xb = jnp.clip(x0i, 0, wli - 2)
    d = x0 - xb.astype(jnp.float32)
    d0 = (d == 0.0).astype(jnp.float32)
    dp1 = (d == 1.0).astype(jnp.float32)
    dm1 = (d == -1.0).astype(jnp.float32)
    wx0v = (1.0 - fx) * vx0
    wx1v = fx * vx1
    w_lo = wx0v * d0 + wx1v * dm1
    w_hi = wx1v * d0 + wx0v * dp1

    yc0 = jnp.clip(y0i, 0, hli - 1)
    yc1 = jnp.clip(y0i + 1, 0, hli - 1)

    r = q.shape[0]
    row_in_block = lax.broadcasted_iota(jnp.int32, (r, 1), 0)
    grow = pl.program_id(0) * r + row_in_block
    b8nv = (grow % BS) * (HEADS * NV)

    wy = [(1.0 - fy) * vy0, fy * vy1]
    base = [
        ls + yc0 * wli + xb + b8nv,
        ls + yc1 * wli + xb + b8nv,
    ]

    for h in range(HEADS):
        s = a[:, h * 16:(h + 1) * 16]
        m = jnp.max(s, axis=-1, keepdims=True)
        e = jnp.exp(s - m)
        aw_h = e / jnp.sum(e, axis=-1, keepdims=True)
        hs = slice(h * 16, (h + 1) * 16)
        for c2 in range(2):
            oi_ref[:, h * 32 + c2 * 16:h * 32 + c2 * 16 + 16] = (
                base[c2][:, hs] + h * NV)
            a_y = aw_h * wy[c2][:, hs]
            wlo = h * 64 + c2 * 32
            ow_ref[:, wlo:wlo + 16] = w_lo[:, hs] * a_y
            ow_ref[:, wlo + 16:wlo + 32] = w_hi[:, hs] * a_y


_SC_G = 8


def _sc_gather_body(table_hbm, idx_hbm, w_hbm, out_hbm,
                    idx_a, idx_b, w_a, w_b, rows_a, rows_b, out_a, out_b,
                    sio_a, sio_b, sg_a, sg_b, so_a, so_b):
    info = plsc.get_sparse_core_info()
    nw = info.num_cores * info.num_subcores
    wid = lax.axis_index("s") * info.num_cores + lax.axis_index("c")
    per_w = NOROWS // nw
    ngroups = per_w // _SC_G
    base0 = wid * per_w
    npair = NSAMP // 2
    gn_i = _SC_G * npair
    gn_w = _SC_G * NSAMP

    def fire_idx(g, ibuf, sem):
        pltpu.async_copy(idx_hbm.at[pl.ds((base0 + g * _SC_G) * npair, gn_i)],
                         ibuf, sem)

    def fire_w(g, wbuf, sem):
        pltpu.async_copy(w_hbm.at[pl.ds((base0 + g * _SC_G) * NSAMP, gn_w)],
                         wbuf, sem)

    def wait_iw(ibuf, wbuf, sem):
        pltpu.make_async_copy(idx_hbm.at[pl.ds(0, gn_i)], ibuf, sem).wait()
        pltpu.make_async_copy(w_hbm.at[pl.ds(0, gn_w)], wbuf, sem).wait()

    def compute(g, wbuf, rows, obuf, osem):
        for o in range(_SC_G):
            pa0 = []
            pa1 = []
            for c2 in range(2):
                acc0 = [jnp.zeros((16,), jnp.float32) for _ in range(2)]
                acc1 = [jnp.zeros((16,), jnp.float32) for _ in range(2)]
                wvl = wbuf[pl.ds(o * NSAMP + c2 * 32, 16)]
                wvh = wbuf[pl.ds(o * NSAMP + c2 * 32 + 16, 16)]
                for j in range(16):
                    s = o * npair + c2 * 16 + j
                    wl_ = _lane_splat(wvl, j)
                    wh_ = _lane_splat(wvh, j)
                    ri0 = plsc.bitcast(rows[s, pl.ds(0, CH)], jnp.int32)
                    ri1 = plsc.bitcast(rows[s, pl.ds(CH, CH)], jnp.int32)
                    re0 = plsc.bitcast(jnp.left_shift(ri0, 16), jnp.float32)
                    ro0 = plsc.bitcast(ri0, jnp.float32)
                    re1 = plsc.bitcast(jnp.left_shift(ri1, 16), jnp.float32)
                    ro1 = plsc.bitcast(ri1, jnp.float32)
                    acc0[j % 2] = acc0[j % 2] + wl_ * re0 + wh_ * re1
                    acc1[j % 2] = acc1[j % 2] + wl_ * ro0 + wh_ * ro1
                pa0.append(acc0[0] + acc0[1])
                pa1.append(acc1[0] + acc1[1])
            obuf[o, pl.ds(0, 16)] = pa0[0] + pa0[1]
            obuf[o, pl.ds(16, 16)] = pa1[0] + pa1[1]
        pltpu.async_copy(obuf, out_hbm.at[pl.ds(base0 + g * _SC_G, _SC_G)],
                         osem)

    fire_idx(0, idx_a, sio_a)
    fire_w(0, w_a, sio_a)
    fire_idx(1, idx_b, sio_b)
    fire_w(1, w_b, sio_b)
    wait_iw(idx_a, w_a, sio_a)
    pltpu.async_copy(table_hbm.at[idx_a], rows_a, sg_a)
    wait_iw(idx_b, w_b, sio_b)

    def drain_out(obuf, osem):
        pltpu.make_async_copy(out_hbm.at[pl.ds(0, _SC_G)], obuf, osem).wait()

    def step(gg, carry):
        g0 = gg * 2
        g1 = g0 + 1
        pltpu.make_async_copy(table_hbm.at[idx_a], rows_a, sg_a).wait()

        @pl.when(gg > 0)
        def _():
            wait_iw(idx_b, w_b, sio_b)

        pltpu.async_copy(table_hbm.at[idx_b], rows_b, sg_b)

        @pl.when(g0 + 2 < ngroups)
        def _():
            fire_idx(g0 + 2, idx_a, sio_a)

        @pl.when(gg > 0)
        def _():
            drain_out(out_a, so_a)

        compute(g0, w_a, rows_a, out_a, so_a)

        @pl.when(g0 + 2 < ngroups)
        def _():
            fire_w(g0 + 2, w_a, sio_a)

        pltpu.make_async_copy(table_hbm.at[idx_b], rows_b, sg_b).wait()

        @pl.when(g0 + 2 < ngroups)
        def _():
            wait_iw(idx_a, w_a, sio_a)
            pltpu.async_copy(table_hbm.at[idx_a], rows_a, sg_a)

        @pl.when(gg > 0)
        def _():
            drain_out(out_b, so_b)

        compute(g1, w_b, rows_b, out_b, so_b)

        @pl.when(g1 + 2 < ngroups)
        def _():
            fire_idx(g1 + 2, idx_b, sio_b)
            fire_w(g1 + 2, w_b, sio_b)

        return carry

    lax.fori_loop(0, ngroups // 2, step, 0)
    pltpu.make_async_copy(out_hbm.at[pl.ds(0, _SC_G)], out_a, so_a).wait()
    pltpu.make_async_copy(out_hbm.at[pl.ds(0, _SC_G)], out_b, so_b).wait()


def _lane_splat(vec, j):
    dnums = lax.GatherDimensionNumbers(
        offset_dims=(), collapsed_slice_dims=(0,), start_index_map=(0,))
    idx = jnp.full((16, 1), j, jnp.int32)
    return lax.gather(vec, idx, dnums, (1,),
                      mode=lax.GatherScatterMode.PROMISE_IN_BOUNDS)


def _tc_mm_bias(x, w, b, block_rows):
    n = x.shape[0]
    k = x.shape[1]
    m = w.shape[1]
    grid = (n // block_rows,)
    return pl.pallas_call(
        _mm_bias_body,
        grid=grid,
        in_specs=[
            pl.BlockSpec((block_rows, k), lambda i: (i, 0)),
            pl.BlockSpec((k, m), lambda i: (0, 0)),
            pl.BlockSpec((1, m), lambda i: (0, 0)),
        ],
        out_specs=pl.BlockSpec((block_rows, m), lambda i: (i, 0)),
        out_shape=jax.ShapeDtypeStruct((n, m), jnp.float32),
    )(x, w, b.reshape(1, m))


def _tc_mm_bias_res(x, w, b, res, block_rows):
    n = x.shape[0]
    k = x.shape[1]
    m = w.shape[1]
    grid = (n // block_rows,)
    return pl.pallas_call(
        _mm_bias_res_body,
        grid=grid,
        in_specs=[
            pl.BlockSpec((block_rows, k), lambda i: (i, 0)),
            pl.BlockSpec((k, m), lambda i: (0, 0)),
            pl.BlockSpec((1, m), lambda i: (0, 0)),
            pl.BlockSpec((block_rows, m), lambda i: (i, 0)),
        ],
        out_specs=pl.BlockSpec((block_rows, m), lambda i: (i, 0)),
        out_shape=jax.ShapeDtypeStruct((n, m), jnp.float32),
    )(x, w, b.reshape(1, m), res)


_B_BLOCK = 512


def _tc_sampling(q2, woxT, woyT, waT, box, boy, ba, refx, refy):
    grid = (NROWS_Q // _B_BLOCK,)
    row_spec = lambda m: pl.BlockSpec((_B_BLOCK, m), lambda i: (i, 0))
    const_spec = lambda m: pl.BlockSpec((1, m), lambda i: (0, 0))
    return pl.pallas_call(
        _sampling_body,
        grid=grid,
        in_specs=[
            row_spec(EMBED),
            pl.BlockSpec((EMBED, HLP), lambda i: (0, 0)),
            pl.BlockSpec((EMBED, HLP), lambda i: (0, 0)),
            pl.BlockSpec((EMBED, HLP), lambda i: (0, 0)),
            const_spec(HLP), const_spec(HLP), const_spec(HLP),
            row_spec(HLP), row_spec(HLP),
            const_spec(HLP), const_spec(HLP),
            const_spec(HLP), const_spec(HLP),
            const_spec(HLP),
        ],
        out_specs=[
            pl.BlockSpec((_B_BLOCK, 256), lambda i: (i, 0)),
            pl.BlockSpec((_B_BLOCK, 512), lambda i: (i, 0)),
        ],
        out_shape=[
            jax.ShapeDtypeStruct((NROWS_Q, 256), jnp.int32),
            jax.ShapeDtypeStruct((NROWS_Q, 512), jnp.float32),
        ],
    )(q2, woxT, woyT, waT, box, boy, ba, refx, refy,
      jnp.asarray(_W_COL), jnp.asarray(_H_COL),
      jnp.asarray(_W_COL.astype(np.int32)), jnp.asarray(_H_COL.astype(np.int32)),
      jnp.asarray(_LS_COL))


def _sc_gather(table, idxf, wf):
    info = plsc.get_sparse_core_info()
    mesh = plsc.VectorSubcoreMesh(core_axis_name="c", subcore_axis_name="s")
    run = pl.kernel(
        _sc_gather_body,
        mesh=mesh,
        out_type=jax.ShapeDtypeStruct((NOROWS, CH), jnp.float32),
        scratch_types=[
            pltpu.VMEM((_SC_G * NSAMP // 2,), jnp.int32),
            pltpu.VMEM((_SC_G * NSAMP // 2,), jnp.int32),
            pltpu.VMEM((_SC_G * NSAMP,), jnp.float32),
            pltpu.VMEM((_SC_G * NSAMP,), jnp.float32),
            pltpu.VMEM((_SC_G * NSAMP // 2, 2 * CH), jnp.bfloat16),
            pltpu.VMEM((_SC_G * NSAMP // 2, 2 * CH), jnp.bfloat16),
            pltpu.VMEM((_SC_G, CH), jnp.float32),
            pltpu.VMEM((_SC_G, CH), jnp.float32),
            pltpu.SemaphoreType.DMA,
            pltpu.SemaphoreType.DMA,
            pltpu.SemaphoreType.DMA,
            pltpu.SemaphoreType.DMA,
            pltpu.SemaphoreType.DMA,
            pltpu.SemaphoreType.DMA,
        ],
        compiler_params=pltpu.CompilerParams(
            use_tc_tiling_on_sc=False, needs_layout_passes=False),
    )
    return run(table, idxf, wf)


def kernel(query, value, reference_points, spatial_shapes, level_start_index,
           W_off, b_off, W_attn, b_attn, W_val, b_val, W_out, b_out):
    q2 = query.reshape(NROWS_Q, EMBED)
    v2 = value.reshape(NV * BS, EMBED)

    table = _tc_vproj_bh(jnp.transpose(value, (1, 0, 2)), W_val.T,
                         b_val).reshape(NV * BS * HEADS, CH)
    table2 = _tc_pair(table)

    refp = jnp.transpose(reference_points, (1, 0, 2, 3)).reshape(NROWS_Q, LEVELS, 2)
    refx = jnp.tile(jnp.repeat(refp[:, :, 0], POINTS, axis=1), (1, HEADS))
    refy = jnp.tile(jnp.repeat(refp[:, :, 1], POINTS, axis=1), (1, HEADS))

    woxT = W_off[0::2].T
    woyT = W_off[1::2].T
    box = b_off[0::2].reshape(1, HLP)
    boy = b_off[1::2].reshape(1, HLP)
    waT = W_attn.T
    ba = b_attn.reshape(1, HLP)

    idx8, w8 = _tc_sampling(q2, woxT, woyT, waT, box, boy, ba, refx, refy)
    idxf = idx8.reshape(NOROWS * NSAMP // 2)
    wf = w8.reshape(NOROWS * NSAMP)

    sampled = _sc_gather(table2, idxf, wf)
    sampled = sampled.reshape(NROWS_Q, EMBED)

    perm = np.empty((EMBED,), np.int32)
    for h in range(HEADS):
        perm[h * CH:h * CH + CH // 2] = h * CH + 2 * np.arange(CH // 2)
        perm[h * CH + CH // 2:(h + 1) * CH] = h * CH + 2 * np.arange(CH // 2) + 1
    w_out_t = W_out.T[jnp.asarray(perm)]

    out = _tc_mm_bias_res(sampled, w_out_t, b_out, q2, 1024)
    return out.reshape(NQ, BS, EMBED)


# speedup vs baseline: 1.1127x; 1.1127x over previous
import functools
import math

import jax
import jax.numpy as jnp
import numpy as np
from jax import lax
from jax.experimental import pallas as pl
from jax.experimental.pallas import tpu as pltpu
from jax.experimental.pallas import tpu_sc as plsc

BS = 2
NQ = 8192
EMBED = 256
HEADS = 8
LEVELS = 4
POINTS = 4
CH = EMBED // HEADS
SHAPES = [[128, 128], [64, 64], [32, 32], [16, 16]]
NV = sum(h * w for h, w in SHAPES)
NROWS_Q = NQ * BS
NOROWS = NQ * BS * HEADS
NSAMP = LEVELS * POINTS * 4
HLP = HEADS * LEVELS * POINTS

_lvl_of_col = np.array([(c // POINTS) % LEVELS for c in range(HLP)], np.int32)
_W_COL = np.array([SHAPES[l][1] for l in _lvl_of_col], np.float32)[None, :]
_H_COL = np.array([SHAPES[l][0] for l in _lvl_of_col], np.float32)[None, :]
_areas = [h * w for h, w in SHAPES]
_lstart = np.concatenate([[0], np.cumsum(_areas)[:-1]]).astype(np.int32)
_LS_COL = _lstart[_lvl_of_col][None, :]


def _mm_bias_body(x_ref, w_ref, b_ref, o_ref):
    o_ref[...] = (
        jnp.dot(x_ref[...], w_ref[...], preferred_element_type=jnp.float32)
        + b_ref[...]
    )


def _mm_bias_bf16_body(x_ref, w_ref, b_ref, o_ref):
    o_ref[...] = (
        jnp.dot(x_ref[...], w_ref[...], preferred_element_type=jnp.float32)
        + b_ref[...]
    ).astype(jnp.bfloat16)


def _mm_bias_res_body(x_ref, w_ref, b_ref, r_ref, o_ref):
    o_ref[...] = (
        jnp.dot(x_ref[...], w_ref[...], preferred_element_type=jnp.float32)
        + b_ref[...]
        + r_ref[...]
    )


def _sampling_body(q_ref, wox_ref, woy_ref, wa_ref, box_ref, boy_ref, ba_ref,
                   refx_ref, refy_ref, wl_ref, hl_ref, wli_ref, hli_ref,
                   ls_ref, oi_ref, ow_ref):
    q = q_ref[...]
    wl = wl_ref[...]
    hl = hl_ref[...]
    wli = wli_ref[...]
    hli = hli_ref[...]
    ls = ls_ref[...]

    offx = jnp.dot(q, wox_ref[...], preferred_element_type=jnp.float32) + box_ref[...]
    offy = jnp.dot(q, woy_ref[...], preferred_element_type=jnp.float32) + boy_ref[...]
    a = jnp.dot(q, wa_ref[...], preferred_element_type=jnp.float32) + ba_ref[...]

    x = refx_ref[...] * wl + offx - 0.5
    y = refy_ref[...] * hl + offy - 0.5
    x0 = jnp.floor(x)
    y0 = jnp.floor(y)
    fx = x - x0
    fy = y - y0
    x0i = x0.astype(jnp.int32)
    y0i = y0.astype(jnp.int32)

    vx0 = (x0 >= 0.0) & (x0 <= wl - 1.0)
    vx1 = (x0 + 1.0 >= 0.0) & (x0 + 1.0 <= wl - 1.0)
    vy0 = (y0 >= 0.0) & (y0 <= hl - 1.0)
    vy1 = (y0 + 1.0 >= 0.0) & (y0 + 1.0 <= hl - 1.0)

    xc0 = jnp.clip(x0i, 0, wli - 1)
    xc1 = jnp.clip(x0i + 1, 0, wli - 1)
    yc0 = jnp.clip(y0i, 0, hli - 1)
    yc1 = jnp.clip(y0i + 1, 0, hli - 1)

    r = q.shape[0]
    row_in_block = lax.broadcasted_iota(jnp.int32, (r, 1), 0)
    grow = pl.program_id(0) * r + row_in_block
    b8 = (grow % BS) * HEADS

    wx0 = 1.0 - fx
    wy0 = 1.0 - fy
    corner_w = [
        wx0 * wy0 * (vx0 & vy0).astype(jnp.float32),
        wx0 * fy * (vx0 & vy1).astype(jnp.float32),
        fx * wy0 * (vx1 & vy0).astype(jnp.float32),
        fx * fy * (vx1 & vy1).astype(jnp.float32),
    ]
    corner_i = [
        (ls + yc0 * wli + xc0) * (BS * HEADS) + b8,
        (ls + yc1 * wli + xc0) * (BS * HEADS) + b8,
        (ls + yc0 * wli + xc1) * (BS * HEADS) + b8,
        (ls + yc1 * wli + xc1) * (BS * HEADS) + b8,
    ]

    for h in range(HEADS):
        s = a[:, h * 16:(h + 1) * 16]
        m = jnp.max(s, axis=-1, keepdims=True)
        e = jnp.exp(s - m)
        aw_h = e / jnp.sum(e, axis=-1, keepdims=True)
        for c in range(4):
            lo = h * 64 + c * 16
            oi_ref[:, lo:lo + 16] = corner_i[c][:, h * 16:(h + 1) * 16] + h
            ow_ref[:, lo:lo + 16] = corner_w[c][:, h * 16:(h + 1) * 16] * aw_h


_SC_G = 8


def _sc_gather_body(table_hbm, idx_hbm, w_hbm, out_hbm,
                    idx_a, idx_b, w_a, w_b, rows_a, rows_b, out_a, out_b,
                    sio_a, sio_b, sg_a, sg_b, so_a, so_b):
    info = plsc.get_sparse_core_info()
    nw = info.num_cores * info.num_subcores
    wid = lax.axis_index("s") * info.num_cores + lax.axis_index("c")
    per_w = NOROWS // nw
    ngroups = per_w // _SC_G
    base0 = wid * per_w
    gn = _SC_G * NSAMP

    def fire_idx(g, ibuf, sem):
        pltpu.async_copy(idx_hbm.at[pl.ds((base0 + g * _SC_G) * NSAMP, gn)],
                         ibuf, sem)

    def fire_w(g, wbuf, sem):
        pltpu.async_copy(w_hbm.at[pl.ds((base0 + g * _SC_G) * NSAMP, gn)],
                         wbuf, sem)

    def wait_iw(ibuf, wbuf, sem):
        pltpu.make_async_copy(idx_hbm.at[pl.ds(0, gn)], ibuf, sem).wait()
        pltpu.make_async_copy(w_hbm.at[pl.ds(0, gn)], wbuf, sem).wait()

    def compute(g, wbuf, rows, obuf, osem):
        for o in range(_SC_G):
            pa0 = []
            pa1 = []
            for kc in range(NSAMP // 16):
                acc0 = [jnp.zeros((16,), jnp.float32) for _ in range(2)]
                acc1 = [jnp.zeros((16,), jnp.float32) for _ in range(2)]
                wv = wbuf[pl.ds(o * NSAMP + kc * 16, 16)]
                for j in range(16):
                    s = o * NSAMP + kc * 16 + j
                    ws = _lane_splat(wv, j)
                    ri = plsc.bitcast(rows[s, :], jnp.int32)
                    re = plsc.bitcast(jnp.left_shift(ri, 16), jnp.float32)
                    ro = plsc.bitcast(ri, jnp.float32)
                    acc0[j % 2] = acc0[j % 2] + ws * re
                    acc1[j % 2] = acc1[j % 2] + ws * ro
                pa0.append(acc0[0] + acc0[1])
                pa1.append(acc1[0] + acc1[1])
            obuf[o, pl.ds(0, 16)] = (pa0[0] + pa0[1]) + (pa0[2] + pa0[3])
            obuf[o, pl.ds(16, 16)] = (pa1[0] + pa1[1]) + (pa1[2] + pa1[3])
        pltpu.async_copy(obuf, out_hbm.at[pl.ds(base0 + g * _SC_G, _SC_G)],
                         osem)

    fire_idx(0, idx_a, sio_a)
    fire_w(0, w_a, sio_a)
    fire_idx(1, idx_b, sio_b)
    fire_w(1, w_b, sio_b)
    wait_iw(idx_a, w_a, sio_a)
    pltpu.async_copy(table_hbm.at[idx_a], rows_a, sg_a)
    wait_iw(idx_b, w_b, sio_b)

    def drain_out(obuf, osem):
        pltpu.make_async_copy(out_hbm.at[pl.ds(0, _SC_G)], obuf, osem).wait()

    def step(gg, carry):
        g0 = gg * 2
        g1 = g0 + 1
        pltpu.make_async_copy(table_hbm.at[idx_a], rows_a, sg_a).wait()

        @pl.when(gg > 0)
        def _():
            wait_iw(idx_b, w_b, sio_b)

        pltpu.async_copy(table_hbm.at[idx_b], rows_b, sg_b)

        @pl.when(g0 + 2 < ngroups)
        def _():
            fire_idx(g0 + 2, idx_a, sio_a)

        @pl.when(gg > 0)
        def _():
            drain_out(out_a, so_a)

        compute(g0, w_a, rows_a, out_a, so_a)

        @pl.when(g0 + 2 < ngroups)
        def _():
            fire_w(g0 + 2, w_a, sio_a)

        pltpu.make_async_copy(table_hbm.at[idx_b], rows_b, sg_b).wait()

        @pl.when(g0 + 2 < ngroups)
        def _():
            wait_iw(idx_a, w_a, sio_a)
            pltpu.async_copy(table_hbm.at[idx_a], rows_a, sg_a)

        @pl.when(gg > 0)
        def _():
            drain_out(out_b, so_b)

        compute(g1, w_b, rows_b, out_b, so_b)

        @pl.when(g1 + 2 < ngroups)
        def _():
            fire_idx(g1 + 2, idx_b, sio_b)
            fire_w(g1 + 2, w_b, sio_b)

        return carry

    lax.fori_loop(0, ngroups // 2, step, 0)
    pltpu.make_async_copy(out_hbm.at[pl.ds(0, _SC_G)], out_a, so_a).wait()
    pltpu.make_async_copy(out_hbm.at[pl.ds(0, _SC_G)], out_b, so_b).wait()


def _lane_splat(vec, j):
    dnums = lax.GatherDimensionNumbers(
        offset_dims=(), collapsed_slice_dims=(0,), start_index_map=(0,))
    idx = jnp.full((16, 1), j, jnp.int32)
    return lax.gather(vec, idx, dnums, (1,),
                      mode=lax.GatherScatterMode.PROMISE_IN_BOUNDS)


def _tc_mm_bias(x, w, b, block_rows, out_dtype=jnp.float32):
    n = x.shape[0]
    k = x.shape[1]
    m = w.shape[1]
    grid = (n // block_rows,)
    body = _mm_bias_bf16_body if out_dtype == jnp.bfloat16 else _mm_bias_body
    return pl.pallas_call(
        body,
        grid=grid,
        in_specs=[
            pl.BlockSpec((block_rows, k), lambda i: (i, 0)),
            pl.BlockSpec((k, m), lambda i: (0, 0)),
            pl.BlockSpec((1, m), lambda i: (0, 0)),
        ],
        out_specs=pl.BlockSpec((block_rows, m), lambda i: (i, 0)),
        out_shape=jax.ShapeDtypeStruct((n, m), out_dtype),
    )(x, w, b.reshape(1, m))


def _tc_mm_bias_res(x, w, b, res, block_rows):
    n = x.shape[0]
    k = x.shape[1]
    m = w.shape[1]
    grid = (n // block_rows,)
    return pl.pallas_call(
        _mm_bias_res_body,
        grid=grid,
        in_specs=[
            pl.BlockSpec((block_rows, k), lambda i: (i, 0)),
            pl.BlockSpec((k, m), lambda i: (0, 0)),
            pl.BlockSpec((1, m), lambda i: (0, 0)),
            pl.BlockSpec((block_rows, m), lambda i: (i, 0)),
        ],
        out_specs=pl.BlockSpec((block_rows, m), lambda i: (i, 0)),
        out_shape=jax.ShapeDtypeStruct((n, m), jnp.float32),
    )(x, w, b.reshape(1, m), res)


_B_BLOCK = 512


def _tc_sampling(q2, woxT, woyT, waT, box, boy, ba, refx, refy):
    grid = (NROWS_Q // _B_BLOCK,)
    row_spec = lambda m: pl.BlockSpec((_B_BLOCK, m), lambda i: (i, 0))
    const_spec = lambda m: pl.BlockSpec((1, m), lambda i: (0, 0))
    return pl.pallas_call(
        _sampling_body,
        grid=grid,
        in_specs=[
            row_spec(EMBED),
            pl.BlockSpec((EMBED, HLP), lambda i: (0, 0)),
            pl.BlockSpec((EMBED, HLP), lambda i: (0, 0)),
            pl.BlockSpec((EMBED, HLP), lambda i: (0, 0)),
            const_spec(HLP), const_spec(HLP), const_spec(HLP),
            row_spec(HLP), row_spec(HLP),
            const_spec(HLP), const_spec(HLP),
            const_spec(HLP), const_spec(HLP),
            const_spec(HLP),
        ],
        out_specs=[
            pl.BlockSpec((_B_BLOCK, 512), lambda i: (i, 0)),
            pl.BlockSpec((_B_BLOCK, 512), lambda i: (i, 0)),
        ],
        out_shape=[
            jax.ShapeDtypeStruct((NROWS_Q, 512), jnp.int32),
            jax.ShapeDtypeStruct((NROWS_Q, 512), jnp.float32),
        ],
    )(q2, woxT, woyT, waT, box, boy, ba, refx, refy,
      jnp.asarray(_W_COL), jnp.asarray(_H_COL),
      jnp.asarray(_W_COL.astype(np.int32)), jnp.asarray(_H_COL.astype(np.int32)),
      jnp.asarray(_LS_COL))


def _sc_gather(table, idxf, wf):
    info = plsc.get_sparse_core_info()
    mesh = plsc.VectorSubcoreMesh(core_axis_name="c", subcore_axis_name="s")
    run = pl.kernel(
        _sc_gather_body,
        mesh=mesh,
        out_type=jax.ShapeDtypeStruct((NOROWS, CH), jnp.float32),
        scratch_types=[
            pltpu.VMEM((_SC_G * NSAMP,), jnp.int32),
            pltpu.VMEM((_SC_G * NSAMP,), jnp.int32),
            pltpu.VMEM((_SC_G * NSAMP,), jnp.float32),
            pltpu.VMEM((_SC_G * NSAMP,), jnp.float32),
            pltpu.VMEM((_SC_G * NSAMP, CH), jnp.bfloat16),
            pltpu.VMEM((_SC_G * NSAMP, CH), jnp.bfloat16),
            pltpu.VMEM((_SC_G, CH), jnp.float32),
            pltpu.VMEM((_SC_G, CH), jnp.float32),
            pltpu.SemaphoreType.DMA,
            pltpu.SemaphoreType.DMA,
            pltpu.SemaphoreType.DMA,
            pltpu.SemaphoreType.DMA,
            pltpu.SemaphoreType.DMA,
            pltpu.SemaphoreType.DMA,
        ],
        compiler_params=pltpu.CompilerParams(
            use_tc_tiling_on_sc=False, needs_layout_passes=False),
    )
    return run(table, idxf, wf)


def kernel(query, value, reference_points, spatial_shapes, level_start_index,
           W_off, b_off, W_attn, b_attn, W_val, b_val, W_out, b_out):
    q2 = query.reshape(NROWS_Q, EMBED)
    v2 = value.reshape(NV * BS, EMBED)

    table = _tc_mm_bias(v2, W_val.T, b_val, 1280, out_dtype=jnp.bfloat16)
    table = table.reshape(NV * BS * HEADS, CH)

    refp = jnp.transpose(reference_points, (1, 0, 2, 3)).reshape(NROWS_Q, LEVELS, 2)
    refx = jnp.tile(jnp.repeat(refp[:, :, 0], POINTS, axis=1), (1, HEADS))
    refy = jnp.tile(jnp.repeat(refp[:, :, 1], POINTS, axis=1), (1, HEADS))

    woxT = W_off[0::2].T
    woyT = W_off[1::2].T
    box = b_off[0::2].reshape(1, HLP)
    boy = b_off[1::2].reshape(1, HLP)
    waT = W_attn.T
    ba = b_attn.reshape(1, HLP)

    idx8, w8 = _tc_sampling(q2, woxT, woyT, waT, box, boy, ba, refx, refy)
    idxf = idx8.reshape(NOROWS * NSAMP)
    wf = w8.reshape(NOROWS * NSAMP)

    sampled = _sc_gather(table, idxf, wf)
    sampled = sampled.reshape(NROWS_Q, EMBED)

    perm = np.empty((EMBED,), np.int32)
    for h in range(HEADS):
        perm[h * CH:h * CH + CH // 2] = h * CH + 2 * np.arange(CH // 2)
        perm[h * CH + CH // 2:(h + 1) * CH] = h * CH + 2 * np.arange(CH // 2) + 1
    w_out_t = W_out.T[jnp.asarray(perm)]

    out = _tc_mm_bias_res(sampled, w_out_t, b_out, q2, 1024)
    return out.reshape(NQ, BS, EMBED)
